# Initial kernel scaffold; baseline (speedup 1.0000x reference)
import jax
import jax.numpy as jnp
from jax.experimental import pallas as pl


def kernel(x, edge_index, batch, w0, b0, gamma0, beta0, w1, b1, gamma1, beta1, w2, b2, gamma2, beta2, w3, b3, gamma3, beta3, wc1, bc1, wc2, bc2):
    raise NotImplementedError("write your pallas kernel here")


# speedup vs baseline: 7.3933x; 7.3933x over previous
import functools

import numpy as np
import jax
import jax.numpy as jnp
from jax import lax
from jax.experimental import pallas as pl
from jax.experimental.pallas import tpu as pltpu
from jax.experimental.pallas import tpu_sc as plsc

_N = 10000
_NPAD = 10240
_E = 320000
_HID = 128
_NG = 64
_NW = 32
_EPT = _NPAD
_CH = 128
_NCH = _EPT // _CH
_RPT = _NPAD // 16
_BLK = 1024
_G = _NPAD // _BLK
_INV = float(1.0 / np.sqrt(1.0 + 1e-5))

_mesh = plsc.VectorSubcoreMesh(core_axis_name="c", subcore_axis_name="s")


@functools.partial(
    pl.kernel,
    out_type=jax.ShapeDtypeStruct((_NW, _NPAD), jnp.float32),
    mesh=_mesh,
    scratch_types=[
        pltpu.VMEM((_EPT,), jnp.int32),
        pltpu.VMEM((_NPAD,), jnp.float32),
    ],
    compiler_params=pltpu.CompilerParams(needs_layout_passes=False),
)
def _sc_deg(dstf, out, dv, hist):
    c = lax.axis_index("c")
    s = lax.axis_index("s")
    wid = s * 2 + c

    def _z(r, _):
        hist[pl.ds(r * 16, 16)] = jnp.zeros((16,), jnp.float32)
        return 0

    lax.fori_loop(0, _RPT, _z, 0)
    pltpu.sync_copy(dstf.at[wid], dv)
    ones = jnp.ones((16,), jnp.float32)

    def _h(i, _):
        idx = dv[pl.ds(i * 16, 16)]
        plsc.addupdate_scatter(hist, [idx], ones)
        return 0

    lax.fori_loop(0, _RPT, _h, 0)
    pltpu.sync_copy(hist, out.at[wid])


@functools.partial(
    pl.kernel,
    out_type=jax.ShapeDtypeStruct((2, _NPAD, _HID), jnp.float32),
    mesh=_mesh,
    scratch_types=[
        pltpu.VMEM((_NCH, _CH), jnp.int32),
        pltpu.VMEM((_NCH, _CH), jnp.int32),
        pltpu.VMEM((_CH, _HID), jnp.float32),
        pltpu.VMEM_SHARED((_NPAD, _HID), jnp.float32),
        pltpu.SemaphoreType.DMA,
    ],
    compiler_params=pltpu.CompilerParams(needs_layout_passes=False),
)
def _sc_scatter(hwp, srct, dstt, zblk, out, src_v, dst_v, buf, acc, sem):
    c = lax.axis_index("c")
    s = lax.axis_index("s")
    wid = s * 2 + c
    base = s * _RPT

    pltpu.sync_copy(zblk, buf)

    def _zc(k, _):
        pltpu.sync_copy(buf, acc.at[pl.ds(base + k * _CH, _CH)])
        return 0

    lax.fori_loop(0, _RPT // _CH, _zc, 0)
    pltpu.sync_copy(srct.at[wid], src_v)
    pltpu.sync_copy(dstt.at[wid], dst_v)
    plsc.subcore_barrier()

    def _step(j, _):
        pltpu.async_copy(hwp.at[src_v.at[j]], buf, sem).wait()
        pltpu.sync_copy(buf, acc.at[dst_v.at[j]], add=True)
        return 0

    lax.fori_loop(0, _NCH, _step, 0)
    plsc.subcore_barrier()
    pltpu.sync_copy(acc.at[pl.ds(base, _RPT)], out.at[c, pl.ds(base, _RPT)])


def _tc_prep_body(degp_ref, o_ref):
    deg = jnp.sum(degp_ref[...], axis=0, keepdims=True) + 1.0
    dinv = lax.rsqrt(deg)
    col = lax.broadcasted_iota(jnp.int32, (1, _NPAD), 1)
    o_ref[...] = jnp.where(col < _N, dinv, 0.0)


_tc_prep = pl.pallas_call(
    _tc_prep_body,
    out_shape=jax.ShapeDtypeStruct((1, _NPAD), jnp.float32),
)


def _tc_hwp0_body(x_ref, dinv_ref, w_ref, o_ref):
    o_ref[...] = dinv_ref[...] * jnp.dot(
        x_ref[...], w_ref[...], preferred_element_type=jnp.float32)


_tc_hwp0 = pl.pallas_call(
    _tc_hwp0_body,
    grid=(_G,),
    in_specs=[
        pl.BlockSpec((_BLK, _HID), lambda i: (i, 0)),
        pl.BlockSpec((_BLK, 1), lambda i: (i, 0)),
        pl.BlockSpec((_HID, _HID), lambda i: (0, 0)),
    ],
    out_specs=pl.BlockSpec((_BLK, _HID), lambda i: (i, 0)),
    out_shape=jax.ShapeDtypeStruct((_NPAD, _HID), jnp.float32),
)


def _tc_layer_body(s_ref, hwp_ref, dinv_ref, b_ref, g_ref, be_ref, w_ref, o_ref):
    dinv = dinv_ref[...]
    agg = dinv * (s_ref[0] + s_ref[1] + hwp_ref[...]) + b_ref[...]
    h = jnp.maximum(g_ref[...] * agg * _INV + be_ref[...], 0.0)
    o_ref[...] = dinv * jnp.dot(h, w_ref[...], preferred_element_type=jnp.float32)


_tc_layer = pl.pallas_call(
    _tc_layer_body,
    grid=(_G,),
    in_specs=[
        pl.BlockSpec((2, _BLK, _HID), lambda i: (0, i, 0)),
        pl.BlockSpec((_BLK, _HID), lambda i: (i, 0)),
        pl.BlockSpec((_BLK, 1), lambda i: (i, 0)),
        pl.BlockSpec((1, _HID), lambda i: (0, 0)),
        pl.BlockSpec((1, _HID), lambda i: (0, 0)),
        pl.BlockSpec((1, _HID), lambda i: (0, 0)),
        pl.BlockSpec((_HID, _HID), lambda i: (0, 0)),
    ],
    out_specs=pl.BlockSpec((_BLK, _HID), lambda i: (i, 0)),
    out_shape=jax.ShapeDtypeStruct((_NPAD, _HID), jnp.float32),
)


def _tc_pool_body(s_ref, hwp_ref, dinv_ref, b_ref, g_ref, be_ref, batch_ref,
                  sums_ref, cnt_ref):
    agg = dinv_ref[...] * (s_ref[0] + s_ref[1] + hwp_ref[...]) + b_ref[...]
    h = jnp.maximum(g_ref[...] * agg * _INV + be_ref[...], 0.0)
    seg = lax.broadcasted_iota(jnp.int32, (_NG, _BLK), 0)
    oh = (batch_ref[0] == seg).astype(jnp.float32)

    @pl.when(pl.program_id(0) == 0)
    def _():
        sums_ref[...] = jnp.zeros_like(sums_ref)
        cnt_ref[...] = jnp.zeros_like(cnt_ref)

    sums_ref[...] += jnp.dot(oh, h, preferred_element_type=jnp.float32)
    cnt_ref[...] += jnp.sum(oh, axis=1, keepdims=True)


_tc_pool = pl.pallas_call(
    _tc_pool_body,
    grid=(_G,),
    in_specs=[
        pl.BlockSpec((2, _BLK, _HID), lambda i: (0, i, 0)),
        pl.BlockSpec((_BLK, _HID), lambda i: (i, 0)),
        pl.BlockSpec((_BLK, 1), lambda i: (i, 0)),
        pl.BlockSpec((1, _HID), lambda i: (0, 0)),
        pl.BlockSpec((1, _HID), lambda i: (0, 0)),
        pl.BlockSpec((1, _HID), lambda i: (0, 0)),
        pl.BlockSpec((1, 1, _BLK), lambda i: (i, 0, 0)),
    ],
    out_specs=[
        pl.BlockSpec((_NG, _HID), lambda i: (0, 0)),
        pl.BlockSpec((_NG, 1), lambda i: (0, 0)),
    ],
    out_shape=[
        jax.ShapeDtypeStruct((_NG, _HID), jnp.float32),
        jax.ShapeDtypeStruct((_NG, 1), jnp.float32),
    ],
)


def _tc_head_body(sums_ref, cnt_ref, wc1_ref, bc1_ref, wc2r_ref, bc2_ref, o_ref):
    pooled = sums_ref[...] / jnp.maximum(cnt_ref[...], 1.0)
    z = jnp.maximum(
        jnp.dot(pooled, wc1_ref[...], preferred_element_type=jnp.float32)
        + bc1_ref[...], 0.0)
    o_ref[...] = jnp.sum(z * wc2r_ref[...], axis=1, keepdims=True) + bc2_ref[...]


_tc_head = pl.pallas_call(
    _tc_head_body,
    out_shape=jax.ShapeDtypeStruct((_NG, 1), jnp.float32),
)


def kernel(x, edge_index, batch, w0, b0, gamma0, beta0, w1, b1, gamma1, beta1,
           w2, b2, gamma2, beta2, w3, b3, gamma3, beta3, wc1, bc1, wc2, bc2):
    epw = _E // _NW
    src = edge_index[0].reshape(_NW, epw)
    dst = edge_index[1].reshape(_NW, epw)
    pad = ((0, 0), (0, _EPT - epw))
    srcp = jnp.pad(src, pad, constant_values=_N)
    dstp = jnp.pad(dst, pad, constant_values=_N)
    srct = srcp.reshape(_NW, _NCH, _CH)
    dstt = dstp.reshape(_NW, _NCH, _CH)

    degp = _sc_deg(dstp)
    dinv = _tc_prep(degp).reshape(_NPAD, 1)

    xp = jnp.pad(x, ((0, _NPAD - _N), (0, 0)))
    zblk = jnp.zeros((_CH, _HID), jnp.float32)
    batchp = jnp.pad(batch, (0, _NPAD - _N), constant_values=-1).reshape(_G, 1, _BLK)

    hwp = _tc_hwp0(xp, dinv, w0)
    layers = [(b0, gamma0, beta0), (b1, gamma1, beta1),
              (b2, gamma2, beta2), (b3, gamma3, beta3)]
    wnext = [w1, w2, w3]
    for l in range(4):
        s_part = _sc_scatter(hwp, srct, dstt, zblk)
        b_l, g_l, be_l = (a.reshape(1, _HID) for a in layers[l])
        if l < 3:
            hwp = _tc_layer(s_part, hwp, dinv, b_l, g_l, be_l, wnext[l])
        else:
            sums, cnt = _tc_pool(s_part, hwp, dinv, b_l, g_l, be_l, batchp)

    out2 = _tc_head(sums, cnt, wc1, bc1.reshape(1, _NG), wc2.reshape(1, _NG),
                    bc2.reshape(1, 1))
    return out2[:, 0]


# speedup vs baseline: 8.4808x; 1.1471x over previous
import functools

import numpy as np
import jax
import jax.numpy as jnp
from jax import lax
from jax.experimental import pallas as pl
from jax.experimental.pallas import tpu as pltpu
from jax.experimental.pallas import tpu_sc as plsc

_N = 10000
_NPAD = 10240
_E = 320000
_HID = 128
_NG = 64
_NW = 32
_EPT = _NPAD
_CH = 128
_NCH = _EPT // _CH
_RPT = _NPAD // 16
_BLK = 1024
_G = _NPAD // _BLK
_INV = float(1.0 / np.sqrt(1.0 + 1e-5))
_NCHH = _NCH // 2

_mesh = plsc.VectorSubcoreMesh(core_axis_name="c", subcore_axis_name="s")


@functools.partial(
    pl.kernel,
    out_type=jax.ShapeDtypeStruct((_NW, _NPAD), jnp.float32),
    mesh=_mesh,
    scratch_types=[
        pltpu.VMEM((_EPT,), jnp.int32),
        pltpu.VMEM((_NPAD,), jnp.float32),
    ],
    compiler_params=pltpu.CompilerParams(needs_layout_passes=False),
)
def _sc_deg(dstf, out, dv, hist):
    c = lax.axis_index("c")
    s = lax.axis_index("s")
    wid = s * 2 + c

    def _z(r, _):
        hist[pl.ds(r * 16, 16)] = jnp.zeros((16,), jnp.float32)
        return 0

    lax.fori_loop(0, _RPT, _z, 0)
    pltpu.sync_copy(dstf.at[wid], dv)
    ones = jnp.ones((16,), jnp.float32)

    def _h(i, _):
        idx = dv[pl.ds(i * 16, 16)]
        plsc.addupdate_scatter(hist, [idx], ones)
        return 0

    lax.fori_loop(0, _RPT, _h, 0)
    pltpu.sync_copy(hist, out.at[wid])


@functools.partial(
    pl.kernel,
    out_type=jax.ShapeDtypeStruct((2, _NPAD, _HID), jnp.float32),
    mesh=_mesh,
    scratch_types=[
        pltpu.VMEM((_NCHH, _CH), jnp.int32),
        pltpu.VMEM((_NCHH, _CH), jnp.int32),
        pltpu.VMEM((_CH, _HID), jnp.float32),
        pltpu.VMEM((_CH, _HID), jnp.float32),
        pltpu.VMEM_SHARED((_NPAD, _HID), jnp.float32),
        pltpu.SemaphoreType.DMA((2,)),
        pltpu.SemaphoreType.DMA((2,)),
    ],
    compiler_params=pltpu.CompilerParams(needs_layout_passes=False),
)
def _sc_scatter(hwp, srct, dstt, zblk, out, src_v, dst_v, buf0, buf1,
                acc, semg, sems):
    bufs = [buf0, buf1]
    c = lax.axis_index("c")
    s = lax.axis_index("s")
    wid = s * 2 + c
    base = s * _RPT

    pltpu.sync_copy(zblk, bufs[0])

    def _zc(k, _):
        pltpu.sync_copy(bufs[0], acc.at[pl.ds(base + k * _CH, _CH)])
        return 0

    lax.fori_loop(0, _RPT // _CH, _zc, 0)
    plsc.subcore_barrier()

    for p in range(_NCH // _NCHH):
        pltpu.sync_copy(srct.at[wid, pl.ds(p * _NCHH, _NCHH)], src_v)
        pltpu.sync_copy(dstt.at[wid, pl.ds(p * _NCHH, _NCHH)], dst_v)
        pltpu.async_copy(hwp.at[src_v.at[0]], bufs[0], semg.at[0])

        def _outer(jo, _):
            for b in range(2):
                j = jo * 2 + b
                bo = (b + 1) % 2

                @pl.when(j + 1 < _NCHH)
                def _fire():
                    @pl.when(j >= 1)
                    def _drain():
                        pltpu.make_async_copy(
                            bufs[bo], acc.at[dst_v.at[j - 1]], sems.at[bo]
                        ).wait()

                    pltpu.async_copy(hwp.at[src_v.at[j + 1]], bufs[bo],
                                     semg.at[bo])

                pltpu.make_async_copy(
                    hwp.at[src_v.at[j]], bufs[b], semg.at[b]).wait()
                pltpu.async_copy(bufs[b], acc.at[dst_v.at[j]], sems.at[b],
                                 add=True)
            return 0

        lax.fori_loop(0, _NCHH // 2, _outer, 0)
        for jj in (_NCHH - 2, _NCHH - 1):
            pltpu.make_async_copy(bufs[jj % 2], acc.at[dst_v.at[jj]],
                                  sems.at[jj % 2]).wait()
    plsc.subcore_barrier()
    pltpu.sync_copy(acc.at[pl.ds(base, _RPT)], out.at[c, pl.ds(base, _RPT)])


def _tc_prep_body(degp_ref, o_ref):
    deg = jnp.sum(degp_ref[...], axis=0, keepdims=True) + 1.0
    dinv = lax.rsqrt(deg)
    col = lax.broadcasted_iota(jnp.int32, (1, _NPAD), 1)
    o_ref[...] = jnp.where(col < _N, dinv, 0.0)


_tc_prep = pl.pallas_call(
    _tc_prep_body,
    out_shape=jax.ShapeDtypeStruct((1, _NPAD), jnp.float32),
)


def _tc_hwp0_body(x_ref, dinv_ref, w_ref, o_ref):
    o_ref[...] = dinv_ref[...] * jnp.dot(
        x_ref[...], w_ref[...], preferred_element_type=jnp.float32)


_tc_hwp0 = pl.pallas_call(
    _tc_hwp0_body,
    grid=(_G,),
    in_specs=[
        pl.BlockSpec((_BLK, _HID), lambda i: (i, 0)),
        pl.BlockSpec((_BLK, 1), lambda i: (i, 0)),
        pl.BlockSpec((_HID, _HID), lambda i: (0, 0)),
    ],
    out_specs=pl.BlockSpec((_BLK, _HID), lambda i: (i, 0)),
    out_shape=jax.ShapeDtypeStruct((_NPAD, _HID), jnp.float32),
)


def _tc_layer_body(s_ref, hwp_ref, dinv_ref, b_ref, g_ref, be_ref, w_ref, o_ref):
    dinv = dinv_ref[...]
    agg = dinv * (s_ref[0] + s_ref[1] + hwp_ref[...]) + b_ref[...]
    h = jnp.maximum(g_ref[...] * agg * _INV + be_ref[...], 0.0)
    o_ref[...] = dinv * jnp.dot(h, w_ref[...], preferred_element_type=jnp.float32)


_tc_layer = pl.pallas_call(
    _tc_layer_body,
    grid=(_G,),
    in_specs=[
        pl.BlockSpec((2, _BLK, _HID), lambda i: (0, i, 0)),
        pl.BlockSpec((_BLK, _HID), lambda i: (i, 0)),
        pl.BlockSpec((_BLK, 1), lambda i: (i, 0)),
        pl.BlockSpec((1, _HID), lambda i: (0, 0)),
        pl.BlockSpec((1, _HID), lambda i: (0, 0)),
        pl.BlockSpec((1, _HID), lambda i: (0, 0)),
        pl.BlockSpec((_HID, _HID), lambda i: (0, 0)),
    ],
    out_specs=pl.BlockSpec((_BLK, _HID), lambda i: (i, 0)),
    out_shape=jax.ShapeDtypeStruct((_NPAD, _HID), jnp.float32),
)


def _tc_pool_body(s_ref, hwp_ref, dinv_ref, b_ref, g_ref, be_ref, batch_ref,
                  sums_ref, cnt_ref):
    agg = dinv_ref[...] * (s_ref[0] + s_ref[1] + hwp_ref[...]) + b_ref[...]
    h = jnp.maximum(g_ref[...] * agg * _INV + be_ref[...], 0.0)
    seg = lax.broadcasted_iota(jnp.int32, (_NG, _BLK), 0)
    oh = (batch_ref[0] == seg).astype(jnp.float32)

    @pl.when(pl.program_id(0) == 0)
    def _():
        sums_ref[...] = jnp.zeros_like(sums_ref)
        cnt_ref[...] = jnp.zeros_like(cnt_ref)

    sums_ref[...] += jnp.dot(oh, h, preferred_element_type=jnp.float32)
    cnt_ref[...] += jnp.sum(oh, axis=1, keepdims=True)


_tc_pool = pl.pallas_call(
    _tc_pool_body,
    grid=(_G,),
    in_specs=[
        pl.BlockSpec((2, _BLK, _HID), lambda i: (0, i, 0)),
        pl.BlockSpec((_BLK, _HID), lambda i: (i, 0)),
        pl.BlockSpec((_BLK, 1), lambda i: (i, 0)),
        pl.BlockSpec((1, _HID), lambda i: (0, 0)),
        pl.BlockSpec((1, _HID), lambda i: (0, 0)),
        pl.BlockSpec((1, _HID), lambda i: (0, 0)),
        pl.BlockSpec((1, 1, _BLK), lambda i: (i, 0, 0)),
    ],
    out_specs=[
        pl.BlockSpec((_NG, _HID), lambda i: (0, 0)),
        pl.BlockSpec((_NG, 1), lambda i: (0, 0)),
    ],
    out_shape=[
        jax.ShapeDtypeStruct((_NG, _HID), jnp.float32),
        jax.ShapeDtypeStruct((_NG, 1), jnp.float32),
    ],
)


def _tc_head_body(sums_ref, cnt_ref, wc1_ref, bc1_ref, wc2r_ref, bc2_ref, o_ref):
    pooled = sums_ref[...] / jnp.maximum(cnt_ref[...], 1.0)
    z = jnp.maximum(
        jnp.dot(pooled, wc1_ref[...], preferred_element_type=jnp.float32)
        + bc1_ref[...], 0.0)
    o_ref[...] = jnp.sum(z * wc2r_ref[...], axis=1, keepdims=True) + bc2_ref[...]


_tc_head = pl.pallas_call(
    _tc_head_body,
    out_shape=jax.ShapeDtypeStruct((_NG, 1), jnp.float32),
)


def kernel(x, edge_index, batch, w0, b0, gamma0, beta0, w1, b1, gamma1, beta1,
           w2, b2, gamma2, beta2, w3, b3, gamma3, beta3, wc1, bc1, wc2, bc2):
    epw = _E // _NW
    src = edge_index[0].reshape(_NW, epw)
    dst = edge_index[1].reshape(_NW, epw)
    pad = ((0, 0), (0, _EPT - epw))
    srcp = jnp.pad(src, pad, constant_values=_N)
    dstp = jnp.pad(dst, pad, constant_values=_N)
    srct = srcp.reshape(_NW, _NCH, _CH)
    dstt = dstp.reshape(_NW, _NCH, _CH)

    degp = _sc_deg(dstp)
    dinv = _tc_prep(degp).reshape(_NPAD, 1)

    xp = jnp.pad(x, ((0, _NPAD - _N), (0, 0)))
    zblk = jnp.zeros((_CH, _HID), jnp.float32)
    batchp = jnp.pad(batch, (0, _NPAD - _N), constant_values=-1).reshape(_G, 1, _BLK)

    hwp = _tc_hwp0(xp, dinv, w0)
    layers = [(b0, gamma0, beta0), (b1, gamma1, beta1),
              (b2, gamma2, beta2), (b3, gamma3, beta3)]
    wnext = [w1, w2, w3]
    for l in range(4):
        s_part = _sc_scatter(hwp, srct, dstt, zblk)
        b_l, g_l, be_l = (a.reshape(1, _HID) for a in layers[l])
        if l < 3:
            hwp = _tc_layer(s_part, hwp, dinv, b_l, g_l, be_l, wnext[l])
        else:
            sums, cnt = _tc_pool(s_part, hwp, dinv, b_l, g_l, be_l, batchp)

    out2 = _tc_head(sums, cnt, wc1, bc1.reshape(1, _NG), wc2.reshape(1, _NG),
                    bc2.reshape(1, 1))
    return out2[:, 0]
